# baseline (speedup 1.0000x reference)
import jax
import jax.numpy as jnp
from jax.experimental import pallas as pl
from jax.experimental.pallas import tpu as pltpu

M_REAL = 5000
MP = 5120
CHUNK = 128
NCH = MP // CHUNK
PW = 256
NBITS = MP // PW
CONV_THR = 0.1
MIN_SZ = 3
GRID_HW = 32
BIG = 1 << 30
SIGMA2x2 = 2.0 * 2.0 * 2.0


def _iou_chunk(boxesR_ref, boxesC_ref, rc):
    i = rc * CHUNK
    cxr = boxesR_ref[0, pl.ds(i, CHUNK), 0:1]
    cyr = boxesR_ref[0, pl.ds(i, CHUNK), 1:2]
    wr = boxesR_ref[0, pl.ds(i, CHUNK), 2:3]
    hr = boxesR_ref[0, pl.ds(i, CHUNK), 3:4]
    cxc = boxesC_ref[0, 0:1, :]
    cyc = boxesC_ref[0, 1:2, :]
    wc = boxesC_ref[0, 2:3, :]
    hc = boxesC_ref[0, 3:4, :]
    x1r = cxr - wr / 2
    y1r = cyr - hr / 2
    x2r = cxr + wr / 2
    y2r = cyr + hr / 2
    x1c = cxc - wc / 2
    y1c = cyc - hc / 2
    x2c = cxc + wc / 2
    y2c = cyc + hc / 2
    ix1 = jnp.maximum(x1r, x1c)
    iy1 = jnp.maximum(y1r, y1c)
    ix2 = jnp.minimum(x2r, x2c)
    iy2 = jnp.minimum(y2r, y2c)
    iw = jnp.clip(ix2 - ix1, 0.0, None)
    ih = jnp.clip(iy2 - iy1, 0.0, None)
    inter = iw * ih
    union = wr * hr + wc * hc - inter
    return inter / jnp.maximum(union, 1e-6)


def _main_kernel(boxesR_ref, boxesC_ref, boxes9_ref, mapw_ref,
                 out8_ref, heat_ref,
                 adj, key_r, key_c, val_r, val_c, nbmax_r, nbmax_c,
                 counts_c, diag_c, seedkey_c, valid_r, centers5,
                 accidx_c, accvalid_c):
    iota_r = jax.lax.broadcasted_iota(jnp.int32, (MP, 1), 0)
    iota_c = jax.lax.broadcasted_iota(jnp.int32, (1, MP), 1)

    counts_c[...] = jnp.zeros((1, MP), jnp.int32)
    diag_c[...] = jnp.zeros((1, MP), jnp.int32)

    def p1(rc, _):
        i = rc * CHUNK
        iou = _iou_chunk(boxesR_ref, boxesC_ref, rc)
        adjb = iou > CONV_THR
        row_ids = i + jax.lax.broadcasted_iota(jnp.int32, (CHUNK, 1), 0)
        diagm = row_ids == iota_c
        adj_ns = jnp.logical_and(adjb, jnp.logical_not(diagm))
        packed = jnp.zeros((CHUNK, PW), jnp.int32)
        for b in range(NBITS):
            packed = packed | jnp.where(
                adj_ns[:, b * PW:(b + 1) * PW], 1 << b, 0)
        adj[pl.ds(i, CHUNK), :] = packed
        adjb_i = jnp.where(adjb, 1, 0)
        cnt = jnp.sum(adjb_i, axis=1, keepdims=True)
        k = cnt * 8192 + (8191 - (i + jax.lax.broadcasted_iota(
            jnp.int32, (CHUNK, 1), 0)))
        key_r[pl.ds(i, CHUNK), :] = k
        val_r[pl.ds(i, CHUNK), :] = k
        counts_c[...] = counts_c[...] + jnp.sum(adjb_i, axis=0, keepdims=True)
        diag_c[...] = diag_c[...] + jnp.sum(
            jnp.where(diagm, adjb_i, 0), axis=0, keepdims=True)
        return 0

    jax.lax.fori_loop(0, NCH, p1, 0)
    kc = counts_c[...] * 8192 + (8191 - iota_c)
    key_c[...] = kc
    val_c[...] = kc

    def round_cond(carry):
        cont, r = carry
        return jnp.logical_and(cont, r < 20480)

    def upd(val, key, nbm):
        alive = jnp.logical_and(val > 0, val < BIG)
        newf = jnp.logical_and(alive, key > nbm)
        dead = jnp.logical_and(alive, nbm >= BIG)
        return jnp.where(newf, BIG, jnp.where(dead, 0, val))

    def round_body(carry):
        _, r = carry
        even = (r % 2) == 0

        @pl.when(even)
        def _():
            vc = val_c[...]

            def sweep_r(rc, _):
                i = rc * CHUNK
                vrchunk = val_r[pl.ds(i, CHUNK), :]
                has_alive = jnp.max(jnp.where(
                    jnp.logical_and(vrchunk > 0, vrchunk < BIG), 1, 0)) > 0

                @pl.when(has_alive)
                def _():
                    apc = adj[pl.ds(i, CHUNK), :]
                    acc = jnp.zeros((CHUNK, PW), jnp.int32)
                    for b in range(NBITS):
                        m = (apc << (31 - b)) < 0
                        acc = jnp.maximum(
                            acc, jnp.where(m, vc[0:1, b * PW:(b + 1) * PW], 0))
                    nbmax_r[pl.ds(i, CHUNK), :] = jnp.max(
                        acc, axis=1, keepdims=True)
                return 0

            jax.lax.fori_loop(0, NCH, sweep_r, 0)
            val_r[...] = upd(val_r[...], key_r[...], nbmax_r[...])

        @pl.when(jnp.logical_not(even))
        def _():
            nbmax_c[...] = jnp.zeros((1, MP), jnp.int32)

            def sweep_c(rc, _):
                i = rc * CHUNK
                vr = val_r[pl.ds(i, CHUNK), :]

                @pl.when(jnp.max(vr) > 0)
                def _():
                    apc = adj[pl.ds(i, CHUNK), :]
                    for b in range(NBITS):
                        m = (apc << (31 - b)) < 0
                        colpart = jnp.max(jnp.where(m, vr, 0), axis=0,
                                          keepdims=True)
                        nbmax_c[0:1, b * PW:(b + 1) * PW] = jnp.maximum(
                            nbmax_c[0:1, b * PW:(b + 1) * PW], colpart)
                return 0

            jax.lax.fori_loop(0, NCH, sweep_c, 0)
            val_c[...] = upd(val_c[...], key_c[...], nbmax_c[...])

        vrn = val_r[...]
        vcn = val_c[...]
        alive_r = jnp.max(jnp.where(
            jnp.logical_and(vrn > 0, vrn < BIG), 1, 0)) > 0
        alive_c = jnp.max(jnp.where(
            jnp.logical_and(vcn > 0, vcn < BIG), 1, 0)) > 0
        return jnp.logical_or(alive_r, alive_c), r + 1

    jax.lax.while_loop(round_cond, round_body, (jnp.bool_(True), jnp.int32(0)))

    seedkey_c[...] = jnp.zeros((1, MP), jnp.int32)

    def p3(rc, _):
        i = rc * CHUNK
        apc = adj[pl.ds(i, CHUNK), :]
        fk = jnp.where(val_r[pl.ds(i, CHUNK), :] >= BIG,
                       key_r[pl.ds(i, CHUNK), :], 0)
        for b in range(NBITS):
            m = (apc << (31 - b)) < 0
            colpart = jnp.max(jnp.where(m, fk, 0), axis=0, keepdims=True)
            seedkey_c[0:1, b * PW:(b + 1) * PW] = jnp.maximum(
                seedkey_c[0:1, b * PW:(b + 1) * PW], colpart)
        return 0

    jax.lax.fori_loop(0, NCH, p3, 0)
    self_k = jnp.where(
        jnp.logical_and(diag_c[...] > 0, val_c[...] >= BIG), key_c[...], 0)
    seedkey_c[...] = jnp.maximum(seedkey_c[...], self_k)

    b9 = boxes9_ref[0]
    skc = seedkey_c[...]

    def p4(rc, _):
        i = rc * CHUNK
        kr = key_r[pl.ds(i, CHUNK), :]
        eq = (skc == kr).astype(jnp.float32)
        csum = jnp.dot(eq, b9, preferred_element_type=jnp.float32,
                       precision=jax.lax.Precision.HIGHEST)
        sizes = csum[:, 4:5]
        valid = sizes >= MIN_SZ
        inv = 1.0 / jnp.maximum(sizes, 1.0)
        ctr = jnp.where(valid, csum[:, 0:4] * inv, 0.0)
        score = jnp.where(valid, sizes / float(M_REAL), 0.0)
        row8 = jnp.concatenate(
            [ctr, score, jnp.zeros((CHUNK, 3), jnp.float32)], axis=1)
        centers5[pl.ds(i, CHUNK), :] = row8
        vb = jnp.logical_and(valid, sizes >= 0.0)
        kr = key_r[pl.ds(i, CHUNK), :]
        accidx_c[...] = accidx_c[...] + jnp.sum(
            jnp.where(jnp.logical_and(vb, kr > kcc), 1, 0),
            axis=0, keepdims=True)
        accvalid_c[...] = accvalid_c[...] + jnp.sum(
            jnp.where(jnp.logical_and(vb, kr == kcc), 1, 0),
            axis=0, keepdims=True)
        return 0

    accidx_c[...] = jnp.zeros((1, MP), jnp.int32)
    accvalid_c[...] = jnp.zeros((1, MP), jnp.int32)
    kcc = key_c[...]
    jax.lax.fori_loop(0, NCH, p4, 0)

    oic = accidx_c[...]
    ovc = accvalid_c[...] > 0
    c5 = centers5[...]

    def p6(pc, _):
        p = pc * CHUNK
        prows = p + jax.lax.broadcasted_iota(jnp.int32, (CHUNK, 1), 0)
        sel = jnp.logical_and(oic == prows, ovc).astype(jnp.float32)
        out8_ref[0, pl.ds(p, CHUNK), :] = jnp.dot(
            sel, c5, preferred_element_type=jnp.float32,
            precision=jax.lax.Precision.HIGHEST)
        return 0

    jax.lax.fori_loop(0, NCH, p6, 0)

    mapw = mapw_ref[0, 0, 0]
    maph = mapw_ref[0, 0, 1]
    cy_c = boxesC_ref[0, 1:2, :]
    cx_r = boxesR_ref[0, :, 0:1]
    hx_r = jnp.clip((cx_r * mapw).astype(jnp.int32), 0,
                    (mapw - 1.0).astype(jnp.int32)).astype(jnp.float32)
    hy_c = jnp.clip((cy_c * maph).astype(jnp.int32), 0,
                    (maph - 1.0).astype(jnp.int32)).astype(jnp.float32)
    ys = jax.lax.broadcasted_iota(jnp.int32, (GRID_HW, 1), 0).astype(jnp.float32)
    xs = jax.lax.broadcasted_iota(jnp.int32, (1, GRID_HW), 1).astype(jnp.float32)
    gyT = jnp.where(iota_c < M_REAL,
                    jnp.exp(-((ys - hy_c) ** 2) / SIGMA2x2), 0.0)
    gx = jnp.exp(-((xs - hx_r) ** 2) / SIGMA2x2)
    heat_ref[0] = jnp.dot(gyT, gx, preferred_element_type=jnp.float32,
                          precision=jax.lax.Precision.HIGHEST) / float(M_REAL)


def kernel(all_boxes, img_h, img_w):
    B, K, N, _ = all_boxes.shape
    boxes = all_boxes.reshape(B, K * N, 4)
    pad = MP - K * N
    boxesR = jnp.pad(boxes, ((0, 0), (0, pad), (0, 0)))
    boxesC = jnp.transpose(boxesR, (0, 2, 1))
    ones = jnp.ones((B, MP, 1), jnp.float32)
    zeros = jnp.zeros((B, MP, 3), jnp.float32)
    boxes9 = jnp.concatenate([boxesR, ones, zeros], axis=2)
    mapw = (img_w // 16).astype(jnp.float32) if hasattr(img_w, "astype") \
        else jnp.float32(img_w // 16)
    maph = (img_h // 16).astype(jnp.float32) if hasattr(img_h, "astype") \
        else jnp.float32(img_h // 16)
    mapwh = jnp.broadcast_to(
        jnp.stack([jnp.asarray(mapw, jnp.float32),
                   jnp.asarray(maph, jnp.float32)]).reshape(1, 1, 2),
        (B, 1, 2))

    out8, heat = pl.pallas_call(
        _main_kernel,
        grid=(B,),
        compiler_params=pltpu.CompilerParams(
            dimension_semantics=("parallel",)),
        in_specs=[
            pl.BlockSpec((1, MP, 4), lambda b: (b, 0, 0)),
            pl.BlockSpec((1, 4, MP), lambda b: (b, 0, 0)),
            pl.BlockSpec((1, MP, 8), lambda b: (b, 0, 0)),
            pl.BlockSpec((1, 1, 2), lambda b: (b, 0, 0)),
        ],
        out_specs=[
            pl.BlockSpec((1, MP, 8), lambda b: (b, 0, 0)),
            pl.BlockSpec((1, GRID_HW, GRID_HW), lambda b: (b, 0, 0)),
        ],
        out_shape=[
            jax.ShapeDtypeStruct((B, MP, 8), jnp.float32),
            jax.ShapeDtypeStruct((B, GRID_HW, GRID_HW), jnp.float32),
        ],
        scratch_shapes=[
            pltpu.VMEM((MP, PW), jnp.int32),
            pltpu.VMEM((MP, 1), jnp.int32),
            pltpu.VMEM((1, MP), jnp.int32),
            pltpu.VMEM((MP, 1), jnp.int32),
            pltpu.VMEM((1, MP), jnp.int32),
            pltpu.VMEM((MP, 1), jnp.int32),
            pltpu.VMEM((1, MP), jnp.int32),
            pltpu.VMEM((1, MP), jnp.int32),
            pltpu.VMEM((1, MP), jnp.int32),
            pltpu.VMEM((1, MP), jnp.int32),
            pltpu.VMEM((MP, 1), jnp.int32),
            pltpu.VMEM((MP, 8), jnp.float32),
            pltpu.VMEM((1, MP), jnp.int32),
            pltpu.VMEM((1, MP), jnp.int32),
        ],
    )(boxesR, boxesC, boxes9, mapwh)

    centers = out8[:, :K * N, 0:4]
    scores = out8[:, :K * N, 4]
    return centers, scores, heat


# speedup vs baseline: 1.1715x; 1.1715x over previous
import jax
import jax.numpy as jnp
from jax.experimental import pallas as pl
from jax.experimental.pallas import tpu as pltpu

M_REAL = 5000
MP = 5120
CHUNK = 128
NCH = MP // CHUNK
PW = 256
NBITS = MP // PW
CONV_THR = 0.1
MIN_SZ = 3
GRID_HW = 32
BIG = 1 << 30
SIGMA2x2 = 2.0 * 2.0 * 2.0


def _iou_chunk(boxesR_ref, boxesC_ref, rc):
    i = rc * CHUNK
    cxr = boxesR_ref[0, pl.ds(i, CHUNK), 0:1]
    cyr = boxesR_ref[0, pl.ds(i, CHUNK), 1:2]
    wr = boxesR_ref[0, pl.ds(i, CHUNK), 2:3]
    hr = boxesR_ref[0, pl.ds(i, CHUNK), 3:4]
    cxc = boxesC_ref[0, 0:1, :]
    cyc = boxesC_ref[0, 1:2, :]
    wc = boxesC_ref[0, 2:3, :]
    hc = boxesC_ref[0, 3:4, :]
    x1r = cxr - wr / 2
    y1r = cyr - hr / 2
    x2r = cxr + wr / 2
    y2r = cyr + hr / 2
    x1c = cxc - wc / 2
    y1c = cyc - hc / 2
    x2c = cxc + wc / 2
    y2c = cyc + hc / 2
    ix1 = jnp.maximum(x1r, x1c)
    iy1 = jnp.maximum(y1r, y1c)
    ix2 = jnp.minimum(x2r, x2c)
    iy2 = jnp.minimum(y2r, y2c)
    iw = jnp.clip(ix2 - ix1, 0.0, None)
    ih = jnp.clip(iy2 - iy1, 0.0, None)
    inter = iw * ih
    union = wr * hr + wc * hc - inter
    return inter / jnp.maximum(union, 1e-6)


def _main_kernel(boxesR_ref, boxesC_ref, boxes9_ref, mapw_ref,
                 out8_ref, heat_ref,
                 adj, key_r, key_c, val_r, val_c, nbmax_r, nbmax_c,
                 counts_c, diag_c, seedkey_c, valid_r, centers5,
                 accidx_c, accvalid_c):
    iota_r = jax.lax.broadcasted_iota(jnp.int32, (MP, 1), 0)
    iota_c = jax.lax.broadcasted_iota(jnp.int32, (1, MP), 1)

    counts_c[...] = jnp.zeros((1, MP), jnp.int32)
    diag_c[...] = jnp.zeros((1, MP), jnp.int32)

    def p1(rc, _):
        i = rc * CHUNK
        iou = _iou_chunk(boxesR_ref, boxesC_ref, rc)
        adjb = iou > CONV_THR
        row_ids = i + jax.lax.broadcasted_iota(jnp.int32, (CHUNK, 1), 0)
        diagm = row_ids == iota_c
        adj_ns = jnp.logical_and(adjb, jnp.logical_not(diagm))
        packed = jnp.zeros((CHUNK, PW), jnp.int32)
        for b in range(NBITS):
            packed = packed | jnp.where(
                adj_ns[:, b * PW:(b + 1) * PW], 1 << b, 0)
        adj[pl.ds(i, CHUNK), :] = packed
        adjb_i = jnp.where(adjb, 1, 0)
        cnt = jnp.sum(adjb_i, axis=1, keepdims=True)
        k = cnt * 8192 + (8191 - (i + jax.lax.broadcasted_iota(
            jnp.int32, (CHUNK, 1), 0)))
        key_r[pl.ds(i, CHUNK), :] = k
        val_r[pl.ds(i, CHUNK), :] = k
        counts_c[...] = counts_c[...] + jnp.sum(adjb_i, axis=0, keepdims=True)
        diag_c[...] = diag_c[...] + jnp.sum(
            jnp.where(diagm, adjb_i, 0), axis=0, keepdims=True)
        return 0

    jax.lax.fori_loop(0, NCH, p1, 0)
    kc = counts_c[...] * 8192 + (8191 - iota_c)
    key_c[...] = kc
    val_c[...] = kc

    def round_cond(carry):
        cont, r = carry
        return jnp.logical_and(cont, r < 20480)

    def round_body(carry):
        _, r = carry
        vc = val_c[...]

        def sweep_r(rc, _):
            i = rc * CHUNK
            vrchunk = val_r[pl.ds(i, CHUNK), :]
            has_alive = jnp.max(jnp.where(
                jnp.logical_and(vrchunk > 0, vrchunk < BIG), 1, 0)) > 0

            @pl.when(has_alive)
            def _():
                apc = adj[pl.ds(i, CHUNK), :]
                acc = jnp.zeros((CHUNK, PW), jnp.int32)
                for b in range(NBITS):
                    m = (apc << (31 - b)) < 0
                    acc = jnp.maximum(
                        acc, jnp.where(m, vc[0:1, b * PW:(b + 1) * PW], 0))
                nbmax_r[pl.ds(i, CHUNK), :] = jnp.max(
                    acc, axis=1, keepdims=True)
            return 0

        jax.lax.fori_loop(0, NCH, sweep_r, 0)

        vr = val_r[...]
        alive = jnp.logical_and(vr > 0, vr < BIG)
        newf = jnp.logical_and(alive, key_r[...] > nbmax_r[...])
        valid_r[...] = jnp.where(newf, 1, 0)
        val_r[...] = jnp.where(newf, BIG, vr)

        nbmax_c[...] = jnp.zeros((1, MP), jnp.int32)

        def kill_sweep(rc, _):
            i = rc * CHUNK
            nfc = valid_r[pl.ds(i, CHUNK), :] > 0

            @pl.when(jnp.max(jnp.where(nfc, 1, 0)) > 0)
            def _():
                apc = adj[pl.ds(i, CHUNK), :]
                for b in range(NBITS):
                    m = jnp.logical_and((apc << (31 - b)) < 0, nfc)
                    colpart = jnp.max(jnp.where(m, 1, 0), axis=0,
                                      keepdims=True)
                    nbmax_c[0:1, b * PW:(b + 1) * PW] = jnp.maximum(
                        nbmax_c[0:1, b * PW:(b + 1) * PW], colpart)
            return 0

        jax.lax.fori_loop(0, NCH, kill_sweep, 0)

        killed_r = jnp.transpose(nbmax_c[...], (1, 0))
        vr2 = val_r[...]
        vr3 = jnp.where(jnp.logical_and(killed_r > 0, vr2 < BIG), 0, vr2)
        val_r[...] = vr3
        val_c[...] = jnp.transpose(vr3, (1, 0))
        alive_any = jnp.max(jnp.where(
            jnp.logical_and(vr3 > 0, vr3 < BIG), 1, 0)) > 0
        return alive_any, r + 1

    jax.lax.while_loop(round_cond, round_body, (jnp.bool_(True), jnp.int32(0)))

    seedkey_c[...] = jnp.zeros((1, MP), jnp.int32)

    def p3(rc, _):
        i = rc * CHUNK
        apc = adj[pl.ds(i, CHUNK), :]
        fk = jnp.where(val_r[pl.ds(i, CHUNK), :] >= BIG,
                       key_r[pl.ds(i, CHUNK), :], 0)
        for b in range(NBITS):
            m = (apc << (31 - b)) < 0
            colpart = jnp.max(jnp.where(m, fk, 0), axis=0, keepdims=True)
            seedkey_c[0:1, b * PW:(b + 1) * PW] = jnp.maximum(
                seedkey_c[0:1, b * PW:(b + 1) * PW], colpart)
        return 0

    jax.lax.fori_loop(0, NCH, p3, 0)
    self_k = jnp.where(
        jnp.logical_and(diag_c[...] > 0, val_c[...] >= BIG), key_c[...], 0)
    seedkey_c[...] = jnp.maximum(seedkey_c[...], self_k)

    b9 = boxes9_ref[0]
    skc = seedkey_c[...]

    def p4(rc, _):
        i = rc * CHUNK
        kr = key_r[pl.ds(i, CHUNK), :]
        eq = (skc == kr).astype(jnp.float32)
        csum = jnp.dot(eq, b9, preferred_element_type=jnp.float32,
                       precision=jax.lax.Precision.HIGHEST)
        sizes = csum[:, 4:5]
        valid = sizes >= MIN_SZ
        inv = 1.0 / jnp.maximum(sizes, 1.0)
        ctr = jnp.where(valid, csum[:, 0:4] * inv, 0.0)
        score = jnp.where(valid, sizes / float(M_REAL), 0.0)
        row8 = jnp.concatenate(
            [ctr, score, jnp.zeros((CHUNK, 3), jnp.float32)], axis=1)
        centers5[pl.ds(i, CHUNK), :] = row8
        vb = jnp.logical_and(valid, sizes >= 0.0)
        kr = key_r[pl.ds(i, CHUNK), :]
        accidx_c[...] = accidx_c[...] + jnp.sum(
            jnp.where(jnp.logical_and(vb, kr > kcc), 1, 0),
            axis=0, keepdims=True)
        accvalid_c[...] = accvalid_c[...] + jnp.sum(
            jnp.where(jnp.logical_and(vb, kr == kcc), 1, 0),
            axis=0, keepdims=True)
        return 0

    accidx_c[...] = jnp.zeros((1, MP), jnp.int32)
    accvalid_c[...] = jnp.zeros((1, MP), jnp.int32)
    kcc = key_c[...]
    jax.lax.fori_loop(0, NCH, p4, 0)

    oic = accidx_c[...]
    ovc = accvalid_c[...] > 0
    c5 = centers5[...]

    def p6(pc, _):
        p = pc * CHUNK
        prows = p + jax.lax.broadcasted_iota(jnp.int32, (CHUNK, 1), 0)
        sel = jnp.logical_and(oic == prows, ovc).astype(jnp.float32)
        out8_ref[0, pl.ds(p, CHUNK), :] = jnp.dot(
            sel, c5, preferred_element_type=jnp.float32,
            precision=jax.lax.Precision.HIGHEST)
        return 0

    jax.lax.fori_loop(0, NCH, p6, 0)

    mapw = mapw_ref[0, 0, 0]
    maph = mapw_ref[0, 0, 1]
    cy_c = boxesC_ref[0, 1:2, :]
    cx_r = boxesR_ref[0, :, 0:1]
    hx_r = jnp.clip((cx_r * mapw).astype(jnp.int32), 0,
                    (mapw - 1.0).astype(jnp.int32)).astype(jnp.float32)
    hy_c = jnp.clip((cy_c * maph).astype(jnp.int32), 0,
                    (maph - 1.0).astype(jnp.int32)).astype(jnp.float32)
    ys = jax.lax.broadcasted_iota(jnp.int32, (GRID_HW, 1), 0).astype(jnp.float32)
    xs = jax.lax.broadcasted_iota(jnp.int32, (1, GRID_HW), 1).astype(jnp.float32)
    gyT = jnp.where(iota_c < M_REAL,
                    jnp.exp(-((ys - hy_c) ** 2) / SIGMA2x2), 0.0)
    gx = jnp.exp(-((xs - hx_r) ** 2) / SIGMA2x2)
    heat_ref[0] = jnp.dot(gyT, gx, preferred_element_type=jnp.float32,
                          precision=jax.lax.Precision.HIGHEST) / float(M_REAL)


def kernel(all_boxes, img_h, img_w):
    B, K, N, _ = all_boxes.shape
    boxes = all_boxes.reshape(B, K * N, 4)
    pad = MP - K * N
    boxesR = jnp.pad(boxes, ((0, 0), (0, pad), (0, 0)))
    boxesC = jnp.transpose(boxesR, (0, 2, 1))
    ones = jnp.ones((B, MP, 1), jnp.float32)
    zeros = jnp.zeros((B, MP, 3), jnp.float32)
    boxes9 = jnp.concatenate([boxesR, ones, zeros], axis=2)
    mapw = (img_w // 16).astype(jnp.float32) if hasattr(img_w, "astype") \
        else jnp.float32(img_w // 16)
    maph = (img_h // 16).astype(jnp.float32) if hasattr(img_h, "astype") \
        else jnp.float32(img_h // 16)
    mapwh = jnp.broadcast_to(
        jnp.stack([jnp.asarray(mapw, jnp.float32),
                   jnp.asarray(maph, jnp.float32)]).reshape(1, 1, 2),
        (B, 1, 2))

    out8, heat = pl.pallas_call(
        _main_kernel,
        grid=(B,),
        compiler_params=pltpu.CompilerParams(
            dimension_semantics=("parallel",)),
        in_specs=[
            pl.BlockSpec((1, MP, 4), lambda b: (b, 0, 0)),
            pl.BlockSpec((1, 4, MP), lambda b: (b, 0, 0)),
            pl.BlockSpec((1, MP, 8), lambda b: (b, 0, 0)),
            pl.BlockSpec((1, 1, 2), lambda b: (b, 0, 0)),
        ],
        out_specs=[
            pl.BlockSpec((1, MP, 8), lambda b: (b, 0, 0)),
            pl.BlockSpec((1, GRID_HW, GRID_HW), lambda b: (b, 0, 0)),
        ],
        out_shape=[
            jax.ShapeDtypeStruct((B, MP, 8), jnp.float32),
            jax.ShapeDtypeStruct((B, GRID_HW, GRID_HW), jnp.float32),
        ],
        scratch_shapes=[
            pltpu.VMEM((MP, PW), jnp.int32),
            pltpu.VMEM((MP, 1), jnp.int32),
            pltpu.VMEM((1, MP), jnp.int32),
            pltpu.VMEM((MP, 1), jnp.int32),
            pltpu.VMEM((1, MP), jnp.int32),
            pltpu.VMEM((MP, 1), jnp.int32),
            pltpu.VMEM((1, MP), jnp.int32),
            pltpu.VMEM((1, MP), jnp.int32),
            pltpu.VMEM((1, MP), jnp.int32),
            pltpu.VMEM((1, MP), jnp.int32),
            pltpu.VMEM((MP, 1), jnp.int32),
            pltpu.VMEM((MP, 8), jnp.float32),
            pltpu.VMEM((1, MP), jnp.int32),
            pltpu.VMEM((1, MP), jnp.int32),
        ],
    )(boxesR, boxesC, boxes9, mapwh)

    centers = out8[:, :K * N, 0:4]
    scores = out8[:, :K * N, 4]
    return centers, scores, heat


# speedup vs baseline: 1.2542x; 1.0707x over previous
import jax
import jax.numpy as jnp
from jax.experimental import pallas as pl
from jax.experimental.pallas import tpu as pltpu

M_REAL = 5000
MP = 5120
CHUNK = 128
NCH = MP // CHUNK
PW = 256
NBITS = MP // PW
CONV_THR = 0.1
MIN_SZ = 3
GRID_HW = 32
BIG = 1 << 30
SIGMA2x2 = 2.0 * 2.0 * 2.0


def _iou_chunk(boxesR_ref, boxesC_ref, rc):
    i = rc * CHUNK
    cxr = boxesR_ref[0, pl.ds(i, CHUNK), 0:1]
    cyr = boxesR_ref[0, pl.ds(i, CHUNK), 1:2]
    wr = boxesR_ref[0, pl.ds(i, CHUNK), 2:3]
    hr = boxesR_ref[0, pl.ds(i, CHUNK), 3:4]
    cxc = boxesC_ref[0, 0:1, :]
    cyc = boxesC_ref[0, 1:2, :]
    wc = boxesC_ref[0, 2:3, :]
    hc = boxesC_ref[0, 3:4, :]
    x1r = cxr - wr / 2
    y1r = cyr - hr / 2
    x2r = cxr + wr / 2
    y2r = cyr + hr / 2
    x1c = cxc - wc / 2
    y1c = cyc - hc / 2
    x2c = cxc + wc / 2
    y2c = cyc + hc / 2
    ix1 = jnp.maximum(x1r, x1c)
    iy1 = jnp.maximum(y1r, y1c)
    ix2 = jnp.minimum(x2r, x2c)
    iy2 = jnp.minimum(y2r, y2c)
    iw = jnp.clip(ix2 - ix1, 0.0, None)
    ih = jnp.clip(iy2 - iy1, 0.0, None)
    inter = iw * ih
    union = wr * hr + wc * hc - inter
    return inter / jnp.maximum(union, 1e-6)


def _main_kernel(boxesR_ref, boxesC_ref, boxes9_ref, mapw_ref,
                 out8_ref, heat_ref,
                 adj, key_r, key_c, val_r, val_c, nbmax_r, nbmax_c,
                 counts_c, diag_c, seedkey_c, valid_r, centers5,
                 accidx_c, accvalid_c):
    iota_r = jax.lax.broadcasted_iota(jnp.int32, (MP, 1), 0)
    iota_c = jax.lax.broadcasted_iota(jnp.int32, (1, MP), 1)

    counts_c[...] = jnp.zeros((1, MP), jnp.int32)
    diag_c[...] = jnp.zeros((1, MP), jnp.int32)

    def p1(rc, _):
        i = rc * CHUNK
        iou = _iou_chunk(boxesR_ref, boxesC_ref, rc)
        adjb = iou > CONV_THR
        row_ids = i + jax.lax.broadcasted_iota(jnp.int32, (CHUNK, 1), 0)
        diagm = row_ids == iota_c
        adj_ns = jnp.logical_and(adjb, jnp.logical_not(diagm))
        packed = jnp.zeros((CHUNK, PW), jnp.int32)
        for b in range(NBITS):
            packed = packed | jnp.where(
                adj_ns[:, b * PW:(b + 1) * PW], 1 << b, 0)
        adj[pl.ds(i, CHUNK), :] = packed
        adjb_i = jnp.where(adjb, 1, 0)
        cnt = jnp.sum(adjb_i, axis=1, keepdims=True)
        k = cnt * 8192 + (8191 - (i + jax.lax.broadcasted_iota(
            jnp.int32, (CHUNK, 1), 0)))
        key_r[pl.ds(i, CHUNK), :] = k
        val_r[pl.ds(i, CHUNK), :] = k
        counts_c[...] = counts_c[...] + jnp.sum(adjb_i, axis=0, keepdims=True)
        diag_c[...] = diag_c[...] + jnp.sum(
            jnp.where(diagm, adjb_i, 0), axis=0, keepdims=True)
        return 0

    jax.lax.fori_loop(0, NCH, p1, 0)
    kc = counts_c[...] * 8192 + (8191 - iota_c)
    key_c[...] = kc
    val_c[...] = kc

    def round_cond(carry):
        cont, r = carry
        return jnp.logical_and(cont, r < 20480)

    def round_body(carry):
        _, r = carry
        vc = val_c[...]

        def sweep_r(rc, _):
            i = rc * CHUNK
            vrchunk = val_r[pl.ds(i, CHUNK), :]
            has_alive = jnp.max(jnp.where(
                jnp.logical_and(vrchunk > 0, vrchunk < BIG), 1, 0)) > 0

            @pl.when(has_alive)
            def _():
                apc = adj[pl.ds(i, CHUNK), :]
                acc = jnp.zeros((CHUNK, PW), jnp.int32)
                for b in range(NBITS):
                    m = (apc << (31 - b)) < 0
                    acc = jnp.maximum(
                        acc, jnp.where(m, vc[0:1, b * PW:(b + 1) * PW], 0))
                nbmax_r[pl.ds(i, CHUNK), :] = jnp.max(
                    acc, axis=1, keepdims=True)
            return 0

        jax.lax.fori_loop(0, NCH, sweep_r, 0)

        vr = val_r[...]
        alive = jnp.logical_and(vr > 0, vr < BIG)
        newf = jnp.logical_and(alive, key_r[...] > nbmax_r[...])
        valid_r[...] = jnp.where(newf, 1, 0)
        val_r[...] = jnp.where(newf, BIG, vr)

        nbmax_c[...] = jnp.zeros((1, MP), jnp.int32)

        def kill_sweep(rc, _):
            i = rc * CHUNK
            nfc = valid_r[pl.ds(i, CHUNK), :] > 0

            @pl.when(jnp.max(jnp.where(nfc, 1, 0)) > 0)
            def _():
                apc = adj[pl.ds(i, CHUNK), :]
                for b in range(NBITS):
                    m = jnp.logical_and((apc << (31 - b)) < 0, nfc)
                    colpart = jnp.max(jnp.where(m, 1, 0), axis=0,
                                      keepdims=True)
                    nbmax_c[0:1, b * PW:(b + 1) * PW] = jnp.maximum(
                        nbmax_c[0:1, b * PW:(b + 1) * PW], colpart)
            return 0

        jax.lax.fori_loop(0, NCH, kill_sweep, 0)

        killed_r = jnp.transpose(nbmax_c[...], (1, 0))
        vr2 = val_r[...]
        vr3 = jnp.where(jnp.logical_and(killed_r > 0, vr2 < BIG), 0, vr2)
        val_r[...] = vr3
        val_c[...] = jnp.transpose(vr3, (1, 0))
        alive_any = jnp.max(jnp.where(
            jnp.logical_and(vr3 > 0, vr3 < BIG), 1, 0)) > 0
        return alive_any, r + 1

    jax.lax.while_loop(round_cond, round_body, (jnp.bool_(True), jnp.int32(0)))

    seedkey_c[...] = jnp.zeros((1, MP), jnp.int32)

    def p3(rc, _):
        i = rc * CHUNK
        apc = adj[pl.ds(i, CHUNK), :]
        fk = jnp.where(val_r[pl.ds(i, CHUNK), :] >= BIG,
                       key_r[pl.ds(i, CHUNK), :], 0)
        for b in range(NBITS):
            m = (apc << (31 - b)) < 0
            colpart = jnp.max(jnp.where(m, fk, 0), axis=0, keepdims=True)
            seedkey_c[0:1, b * PW:(b + 1) * PW] = jnp.maximum(
                seedkey_c[0:1, b * PW:(b + 1) * PW], colpart)
        return 0

    jax.lax.fori_loop(0, NCH, p3, 0)
    self_k = jnp.where(
        jnp.logical_and(diag_c[...] > 0, val_c[...] >= BIG), key_c[...], 0)
    seedkey_c[...] = jnp.maximum(seedkey_c[...], self_k)

    b9 = boxes9_ref[0]
    skc = seedkey_c[...]

    def p4(rc, _):
        i = rc * CHUNK
        kr = key_r[pl.ds(i, CHUNK), :]
        eq = (skc == kr).astype(jnp.float32)
        csum = jnp.dot(eq, b9, preferred_element_type=jnp.float32,
                       precision=jax.lax.Precision.HIGHEST)
        sizes = csum[:, 4:5]
        valid = sizes >= MIN_SZ
        inv = 1.0 / jnp.maximum(sizes, 1.0)
        ctr = jnp.where(valid, csum[:, 0:4] * inv, 0.0)
        score = jnp.where(valid, sizes / float(M_REAL), 0.0)
        row8 = jnp.concatenate(
            [ctr, score, jnp.zeros((CHUNK, 3), jnp.float32)], axis=1)
        centers5[pl.ds(i, CHUNK), :] = row8
        vb = jnp.logical_and(valid, sizes >= 0.0)
        kr = key_r[pl.ds(i, CHUNK), :]
        accidx_c[...] = accidx_c[...] + jnp.sum(
            jnp.where(jnp.logical_and(vb, kr > kcc), 1, 0),
            axis=0, keepdims=True)
        accvalid_c[...] = accvalid_c[...] + jnp.sum(
            jnp.where(jnp.logical_and(vb, kr == kcc), 1, 0),
            axis=0, keepdims=True)
        return 0

    accidx_c[...] = jnp.zeros((1, MP), jnp.int32)
    accvalid_c[...] = jnp.zeros((1, MP), jnp.int32)
    kcc = key_c[...]
    jax.lax.fori_loop(0, NCH, p4, 0)

    oic = accidx_c[...]
    ovc = accvalid_c[...] > 0
    c5 = centers5[...]

    def p6(pc, _):
        p = pc * CHUNK
        prows = p + jax.lax.broadcasted_iota(jnp.int32, (CHUNK, 1), 0)
        sel = jnp.logical_and(oic == prows, ovc).astype(jnp.float32)
        out8_ref[0, pl.ds(p, CHUNK), :] = jnp.dot(
            sel, c5, preferred_element_type=jnp.float32,
            precision=jax.lax.Precision.HIGHEST)
        return 0

    NVCH = 14
    jax.lax.fori_loop(0, NVCH, p6, 0)
    out8_ref[0, pl.ds(NVCH * CHUNK, MP - NVCH * CHUNK), :] = jnp.zeros(
        (MP - NVCH * CHUNK, 8), jnp.float32)

    mapw = mapw_ref[0, 0, 0]
    maph = mapw_ref[0, 0, 1]
    cy_c = boxesC_ref[0, 1:2, :]
    cx_r = boxesR_ref[0, :, 0:1]
    hx_r = jnp.clip((cx_r * mapw).astype(jnp.int32), 0,
                    (mapw - 1.0).astype(jnp.int32)).astype(jnp.float32)
    hy_c = jnp.clip((cy_c * maph).astype(jnp.int32), 0,
                    (maph - 1.0).astype(jnp.int32)).astype(jnp.float32)
    ys = jax.lax.broadcasted_iota(jnp.int32, (GRID_HW, 1), 0).astype(jnp.float32)
    xs = jax.lax.broadcasted_iota(jnp.int32, (1, GRID_HW), 1).astype(jnp.float32)
    gyT = jnp.where(iota_c < M_REAL,
                    jnp.exp(-((ys - hy_c) ** 2) / SIGMA2x2), 0.0)
    gx = jnp.exp(-((xs - hx_r) ** 2) / SIGMA2x2)
    heat_ref[0] = jnp.dot(gyT, gx, preferred_element_type=jnp.float32,
                          precision=jax.lax.Precision.HIGHEST) / float(M_REAL)


def kernel(all_boxes, img_h, img_w):
    B, K, N, _ = all_boxes.shape
    boxes = all_boxes.reshape(B, K * N, 4)
    pad = MP - K * N
    boxesR = jnp.pad(boxes, ((0, 0), (0, pad), (0, 0)))
    boxesC = jnp.transpose(boxesR, (0, 2, 1))
    ones = jnp.ones((B, MP, 1), jnp.float32)
    zeros = jnp.zeros((B, MP, 3), jnp.float32)
    boxes9 = jnp.concatenate([boxesR, ones, zeros], axis=2)
    mapw = (img_w // 16).astype(jnp.float32) if hasattr(img_w, "astype") \
        else jnp.float32(img_w // 16)
    maph = (img_h // 16).astype(jnp.float32) if hasattr(img_h, "astype") \
        else jnp.float32(img_h // 16)
    mapwh = jnp.broadcast_to(
        jnp.stack([jnp.asarray(mapw, jnp.float32),
                   jnp.asarray(maph, jnp.float32)]).reshape(1, 1, 2),
        (B, 1, 2))

    out8, heat = pl.pallas_call(
        _main_kernel,
        grid=(B,),
        compiler_params=pltpu.CompilerParams(
            dimension_semantics=("parallel",)),
        in_specs=[
            pl.BlockSpec((1, MP, 4), lambda b: (b, 0, 0)),
            pl.BlockSpec((1, 4, MP), lambda b: (b, 0, 0)),
            pl.BlockSpec((1, MP, 8), lambda b: (b, 0, 0)),
            pl.BlockSpec((1, 1, 2), lambda b: (b, 0, 0)),
        ],
        out_specs=[
            pl.BlockSpec((1, MP, 8), lambda b: (b, 0, 0)),
            pl.BlockSpec((1, GRID_HW, GRID_HW), lambda b: (b, 0, 0)),
        ],
        out_shape=[
            jax.ShapeDtypeStruct((B, MP, 8), jnp.float32),
            jax.ShapeDtypeStruct((B, GRID_HW, GRID_HW), jnp.float32),
        ],
        scratch_shapes=[
            pltpu.VMEM((MP, PW), jnp.int32),
            pltpu.VMEM((MP, 1), jnp.int32),
            pltpu.VMEM((1, MP), jnp.int32),
            pltpu.VMEM((MP, 1), jnp.int32),
            pltpu.VMEM((1, MP), jnp.int32),
            pltpu.VMEM((MP, 1), jnp.int32),
            pltpu.VMEM((1, MP), jnp.int32),
            pltpu.VMEM((1, MP), jnp.int32),
            pltpu.VMEM((1, MP), jnp.int32),
            pltpu.VMEM((1, MP), jnp.int32),
            pltpu.VMEM((MP, 1), jnp.int32),
            pltpu.VMEM((MP, 8), jnp.float32),
            pltpu.VMEM((1, MP), jnp.int32),
            pltpu.VMEM((1, MP), jnp.int32),
        ],
    )(boxesR, boxesC, boxes9, mapwh)

    centers = out8[:, :K * N, 0:4]
    scores = out8[:, :K * N, 4]
    return centers, scores, heat
